# baseline (speedup 1.0000x reference)
import functools

import jax
import jax.numpy as jnp
from jax import lax
from jax.experimental import pallas as pl
from jax.experimental.pallas import tpu as pltpu
from jax.experimental.pallas import tpu_sc as plsc

N_B2 = 10000
N_B1 = 10000
E = 320000
D = 128
H = 128

NC = 2
NS = 16
LANES = 16
NW = NC * NS
CHUNK = 125
CHUNKS_PER_W = 80
IBLK = 8
NBLK = CHUNKS_PER_W // IBLK
N_ACC = 10240
ROWS_PER_TILE = N_ACC // NS
ZCOPY = 80
ROW_BLK = 512
N_BLOCKS = N_ACC // ROW_BLK


def _sc_segment_sum(x_b2, src_w, dst_w):
  mesh = plsc.VectorSubcoreMesh(core_axis_name="c", subcore_axis_name="s")

  @functools.partial(
      pl.kernel,
      out_type=(
          jax.ShapeDtypeStruct((NC, N_ACC, D), jnp.float32),
          jax.ShapeDtypeStruct((NW, N_ACC), jnp.float32),
      ),
      mesh=mesh,
      compiler_params=pltpu.CompilerParams(needs_layout_passes=False),
      scratch_types=[
          pltpu.VMEM((2, IBLK, CHUNK), jnp.int32),
          pltpu.VMEM((2, IBLK, CHUNK), jnp.int32),
          pltpu.VMEM((CHUNK, D), jnp.float32),
          pltpu.VMEM((CHUNK, D), jnp.float32),
          pltpu.VMEM((N_ACC,), jnp.float32),
          pltpu.VMEM_SHARED((N_ACC, D), jnp.float32),
          pltpu.SemaphoreType.DMA,
          pltpu.SemaphoreType.DMA,
          pltpu.SemaphoreType.DMA,
          pltpu.SemaphoreType.DMA,
          pltpu.SemaphoreType.DMA,
          pltpu.SemaphoreType.DMA,
      ],
  )
  def k(x_hbm, src_hbm, dst_hbm, psum_hbm, pcnt_hbm,
        src_b, dst_b, rows0_v, rows1_v, cnt_v, acc_sh,
        gsem0, gsem1, ssem0, ssem1, dsem0, dsem1):
    rows = (rows0_v, rows1_v)
    gsems = (gsem0, gsem1)
    ssems = (ssem0, ssem1)
    dsems = (dsem0, dsem1)
    c = lax.axis_index("c")
    s = lax.axis_index("s")
    wid = s * NC + c

    zero16 = jnp.zeros((LANES,), jnp.float32)
    one16 = jnp.ones((LANES,), jnp.float32)
    tail_mask = lax.iota(jnp.int32, LANES) >= (LANES - (CHUNK % LANES))

    def zb(t, _):
      rows0_v[t // 8, pl.ds((t % 8) * LANES, LANES)] = zero16
      return 0
    lax.fori_loop(0, CHUNK * 8, zb, 0)

    def zc(t, _):
      cnt_v[pl.ds(t * LANES, LANES)] = zero16
      return 0
    lax.fori_loop(0, N_ACC // LANES, zc, 0)

    for kk in range(ROWS_PER_TILE // ZCOPY):
      pltpu.sync_copy(rows0_v.at[pl.ds(0, ZCOPY)],
                      acc_sh.at[pl.ds(s * ROWS_PER_TILE + kk * ZCOPY, ZCOPY)])
    plsc.subcore_barrier()

    def start_iblk(bb, pb):
      pltpu.async_copy(src_hbm.at[wid, pl.ds(bb * IBLK, IBLK)], src_b.at[pb],
                       ssems[pb])
      pltpu.async_copy(dst_hbm.at[wid, pl.ds(bb * IBLK, IBLK)], dst_b.at[pb],
                       dsems[pb])

    def wait_iblk(pb):
      pltpu.make_async_copy(src_hbm.at[wid, pl.ds(0, IBLK)], src_b.at[pb],
                            ssems[pb]).wait()
      pltpu.make_async_copy(dst_hbm.at[wid, pl.ds(0, IBLK)], dst_b.at[pb],
                            dsems[pb]).wait()

    start_iblk(0, 0)
    wait_iblk(0)
    pltpu.async_copy(x_hbm.at[src_b.at[0, 0]], rows[0], gsems[0])

    def outer(bb2, _):
      for pb in range(2):
        bb = 2 * bb2 + pb
        np_ = 1 - pb

        @pl.when(bb + 1 < NBLK)
        def _():
          start_iblk(bb + 1, np_)

        for jj in range(IBLK):
          b = jj % 2
          nb = 1 - b
          if jj < IBLK - 1:
            pltpu.async_copy(x_hbm.at[src_b.at[pb, jj + 1]], rows[nb],
                             gsems[nb])
          else:
            @pl.when(bb + 1 < NBLK)
            def _():
              wait_iblk(np_)
              pltpu.async_copy(x_hbm.at[src_b.at[np_, 0]], rows[nb],
                               gsems[nb])
          for i in range(CHUNK // LANES):
            idx = dst_b[pb, jj, pl.ds(i * LANES, LANES)]
            plsc.addupdate_scatter(cnt_v, [idx], one16)
          idx = dst_b[pb, jj, pl.ds(CHUNK - LANES, LANES)]
          plsc.addupdate_scatter(cnt_v, [idx], one16, mask=tail_mask)

          pltpu.make_async_copy(x_hbm.at[src_b.at[pb, jj]], rows[b],
                                gsems[b]).wait()
          pltpu.sync_copy(rows[b], acc_sh.at[dst_b.at[pb, jj]], add=True)
      return 0
    lax.fori_loop(0, NBLK // 2, outer, 0)

    plsc.subcore_barrier()

    pltpu.sync_copy(acc_sh.at[pl.ds(s * ROWS_PER_TILE, ROWS_PER_TILE)],
                    psum_hbm.at[c, pl.ds(s * ROWS_PER_TILE, ROWS_PER_TILE)])
    pltpu.sync_copy(cnt_v, pcnt_hbm.at[wid])

  return k(x_b2, src_w, dst_w)


def _tc_body(psum_ref, pcnt_ref, x_ref, wl_ref, wr_ref, bl_ref,
             w1_ref, b1_ref, w2_ref, b2_ref, w3_ref, b3_ref, out_ref):
  summed = psum_ref[0] + psum_ref[1]
  cnt = jnp.sum(pcnt_ref[...], axis=0)
  mean = summed / jnp.maximum(cnt, 1.0)[:, None]
  h = (jnp.dot(mean, wl_ref[...], preferred_element_type=jnp.float32)
       + jnp.dot(x_ref[...], wr_ref[...], preferred_element_type=jnp.float32)
       + bl_ref[...])
  h = jnp.dot(h, w1_ref[...], preferred_element_type=jnp.float32) + b1_ref[...]
  h = jnp.where(h > 0, h, 0.01 * h)
  h = jnp.dot(h, w2_ref[...], preferred_element_type=jnp.float32) + b2_ref[...]
  h = jnp.where(h > 0, h, 0.01 * h)
  out_ref[...] = jnp.dot(h, w3_ref[...], preferred_element_type=jnp.float32) + b3_ref[...]


def kernel(x_B2, x_B1, edge_index, W_l, b_l, W_r, W1, b1, g1, be1, W2, b2, g2, be2, W3, b3):
  src_w = edge_index[0].astype(jnp.int32).reshape(NW, CHUNKS_PER_W, CHUNK)
  dst_w = edge_index[1].astype(jnp.int32).reshape(NW, CHUNKS_PER_W, CHUNK)

  psum, pcnt = _sc_segment_sum(x_B2, src_w, dst_w)

  eps = 1e-5
  s1 = g1 / jnp.sqrt(1.0 + eps)
  s2 = g2 / jnp.sqrt(1.0 + eps)
  w1f = (W1 * s1[:, None]).T
  b1f = b1 * s1 + be1
  w2f = (W2 * s2[:, None]).T
  b2f = b2 * s2 + be2
  w3t = W3.T
  wlt = W_l.T
  wrt = W_r.T

  out = pl.pallas_call(
      _tc_body,
      grid=(N_BLOCKS,),
      in_specs=[
          pl.BlockSpec((NC, ROW_BLK, D), lambda i: (0, i, 0)),
          pl.BlockSpec((NW, ROW_BLK), lambda i: (0, i)),
          pl.BlockSpec((ROW_BLK, D), lambda i: (i, 0)),
          pl.BlockSpec((D, H), lambda i: (0, 0)),
          pl.BlockSpec((D, H), lambda i: (0, 0)),
          pl.BlockSpec((H,), lambda i: (0,)),
          pl.BlockSpec((H, 1280), lambda i: (0, 0)),
          pl.BlockSpec((1280,), lambda i: (0,)),
          pl.BlockSpec((1280, 480), lambda i: (0, 0)),
          pl.BlockSpec((480,), lambda i: (0,)),
          pl.BlockSpec((480, 1), lambda i: (0, 0)),
          pl.BlockSpec((1,), lambda i: (0,)),
      ],
      out_specs=pl.BlockSpec((ROW_BLK, 1), lambda i: (i, 0)),
      out_shape=jax.ShapeDtypeStruct((N_ACC, 1), jnp.float32),
  )(psum, pcnt, x_B1, wlt, wrt, b_l, w1f, b1f, w2f, b2f, w3t, b3)

  return out[:N_B1, 0]


# speedup vs baseline: 1.0163x; 1.0163x over previous
import functools

import jax
import jax.numpy as jnp
from jax import lax
from jax.experimental import pallas as pl
from jax.experimental.pallas import tpu as pltpu
from jax.experimental.pallas import tpu_sc as plsc

N_B2 = 10000
N_B1 = 10000
E = 320000
D = 128
H = 128

NC = 2
NS = 16
LANES = 16
NW = NC * NS
CHUNK = 125
CHUNKS_PER_W = 80
IBLK = 8
NBLK = CHUNKS_PER_W // IBLK
N_ACC = 10240
ROWS_PER_TILE = N_ACC // NS
ZCOPY = 80
ROW_BLK = 512
N_BLOCKS = N_ACC // ROW_BLK


def _sc_segment_sum(x_b2, src_w, dst_w):
  mesh = plsc.VectorSubcoreMesh(core_axis_name="c", subcore_axis_name="s")

  @functools.partial(
      pl.kernel,
      out_type=(
          jax.ShapeDtypeStruct((NC, N_ACC, D), jnp.float32),
          jax.ShapeDtypeStruct((NW, N_ACC), jnp.float32),
      ),
      mesh=mesh,
      compiler_params=pltpu.CompilerParams(needs_layout_passes=False),
      scratch_types=[
          pltpu.VMEM((2, IBLK, CHUNK), jnp.int32),
          pltpu.VMEM((2, IBLK, CHUNK), jnp.int32),
          pltpu.VMEM((CHUNK, D), jnp.float32),
          pltpu.VMEM((CHUNK, D), jnp.float32),
          pltpu.VMEM((N_ACC,), jnp.float32),
          pltpu.VMEM_SHARED((N_ACC, D), jnp.float32),
          pltpu.SemaphoreType.DMA,
          pltpu.SemaphoreType.DMA,
          pltpu.SemaphoreType.DMA,
          pltpu.SemaphoreType.DMA,
          pltpu.SemaphoreType.DMA,
          pltpu.SemaphoreType.DMA,
      ],
  )
  def k(x_hbm, src_hbm, dst_hbm, psum_hbm, pcnt_hbm,
        src_b, dst_b, rows0_v, rows1_v, cnt_v, acc_sh,
        gsem0, gsem1, ssem0, ssem1, dsem0, dsem1):
    rows = (rows0_v, rows1_v)
    gsems = (gsem0, gsem1)
    ssems = (ssem0, ssem1)
    dsems = (dsem0, dsem1)
    c = lax.axis_index("c")
    s = lax.axis_index("s")
    wid = s * NC + c

    def start_iblk(bb, pb):
      pltpu.async_copy(src_hbm.at[wid, pl.ds(bb * IBLK, IBLK)], src_b.at[pb],
                       ssems[pb])
      pltpu.async_copy(dst_hbm.at[wid, pl.ds(bb * IBLK, IBLK)], dst_b.at[pb],
                       dsems[pb])

    def wait_iblk(pb):
      pltpu.make_async_copy(src_hbm.at[wid, pl.ds(0, IBLK)], src_b.at[pb],
                            ssems[pb]).wait()
      pltpu.make_async_copy(dst_hbm.at[wid, pl.ds(0, IBLK)], dst_b.at[pb],
                            dsems[pb]).wait()

    start_iblk(0, 0)

    zero16 = jnp.zeros((LANES,), jnp.float32)
    one16 = jnp.ones((LANES,), jnp.float32)
    tail_mask = lax.iota(jnp.int32, LANES) >= (LANES - (CHUNK % LANES))

    def zb(t, _):
      rows0_v[t // 8, pl.ds((t % 8) * LANES, LANES)] = zero16
      return 0
    lax.fori_loop(0, CHUNK * 8, zb, 0)

    def zc(t, _):
      cnt_v[pl.ds(t * LANES, LANES)] = zero16
      return 0
    lax.fori_loop(0, N_ACC // LANES, zc, 0)

    for kk in range(ROWS_PER_TILE // ZCOPY):
      pltpu.sync_copy(rows0_v.at[pl.ds(0, ZCOPY)],
                      acc_sh.at[pl.ds(s * ROWS_PER_TILE + kk * ZCOPY, ZCOPY)])
    plsc.subcore_barrier()

    wait_iblk(0)
    pltpu.async_copy(x_hbm.at[src_b.at[0, 0]], rows[0], gsems[0])

    def outer(bb2, _):
      for pb in range(2):
        bb = 2 * bb2 + pb
        np_ = 1 - pb

        @pl.when(bb + 1 < NBLK)
        def _():
          start_iblk(bb + 1, np_)

        for jj in range(IBLK):
          b = jj % 2
          nb = 1 - b
          if jj < IBLK - 1:
            pltpu.async_copy(x_hbm.at[src_b.at[pb, jj + 1]], rows[nb],
                             gsems[nb])
          else:
            @pl.when(bb + 1 < NBLK)
            def _():
              wait_iblk(np_)
              pltpu.async_copy(x_hbm.at[src_b.at[np_, 0]], rows[nb],
                               gsems[nb])
          for i in range(CHUNK // LANES):
            idx = dst_b[pb, jj, pl.ds(i * LANES, LANES)]
            plsc.addupdate_scatter(cnt_v, [idx], one16)
          idx = dst_b[pb, jj, pl.ds(CHUNK - LANES, LANES)]
          plsc.addupdate_scatter(cnt_v, [idx], one16, mask=tail_mask)

          pltpu.make_async_copy(x_hbm.at[src_b.at[pb, jj]], rows[b],
                                gsems[b]).wait()
          pltpu.sync_copy(rows[b], acc_sh.at[dst_b.at[pb, jj]], add=True)
      return 0
    lax.fori_loop(0, NBLK // 2, outer, 0)

    plsc.subcore_barrier()

    pltpu.sync_copy(acc_sh.at[pl.ds(s * ROWS_PER_TILE, ROWS_PER_TILE)],
                    psum_hbm.at[c, pl.ds(s * ROWS_PER_TILE, ROWS_PER_TILE)])
    pltpu.sync_copy(cnt_v, pcnt_hbm.at[wid])

  return k(x_b2, src_w, dst_w)


def _tc_body(psum_ref, pcnt_ref, x_ref, wl_ref, wr_ref, bl_ref,
             w1_ref, b1_ref, w2_ref, b2_ref, w3_ref, b3_ref, out_ref):
  summed = psum_ref[0] + psum_ref[1]
  cnt = jnp.sum(pcnt_ref[...], axis=0)
  mean = summed / jnp.maximum(cnt, 1.0)[:, None]
  h = (jnp.dot(mean, wl_ref[...], preferred_element_type=jnp.float32)
       + jnp.dot(x_ref[...], wr_ref[...], preferred_element_type=jnp.float32)
       + bl_ref[...])
  h = jnp.dot(h.astype(jnp.bfloat16), w1_ref[...],
              preferred_element_type=jnp.float32) + b1_ref[...]
  h = jnp.where(h > 0, h, 0.01 * h)
  h = jnp.dot(h.astype(jnp.bfloat16), w2_ref[...],
              preferred_element_type=jnp.float32) + b2_ref[...]
  h = jnp.where(h > 0, h, 0.01 * h)
  out_ref[...] = jnp.sum(h * w3_ref[...], axis=1) + b3_ref[...]


def kernel(x_B2, x_B1, edge_index, W_l, b_l, W_r, W1, b1, g1, be1, W2, b2, g2, be2, W3, b3):
  if edge_index.dtype == jnp.int64:
    ei32 = lax.bitcast_convert_type(edge_index, jnp.int32)[:, :, 0]
  else:
    ei32 = edge_index.astype(jnp.int32)
  src_w = ei32[0].reshape(NW, CHUNKS_PER_W, CHUNK)
  dst_w = ei32[1].reshape(NW, CHUNKS_PER_W, CHUNK)

  psum, pcnt = _sc_segment_sum(x_B2, src_w, dst_w)

  eps = 1e-5
  s1 = g1 / jnp.sqrt(1.0 + eps)
  s2 = g2 / jnp.sqrt(1.0 + eps)
  w1f = (W1 * s1[:, None]).T.astype(jnp.bfloat16)
  b1f = b1 * s1 + be1
  w2f = (W2 * s2[:, None]).T.astype(jnp.bfloat16)
  b2f = b2 * s2 + be2
  w3r = W3[0]
  wlt = W_l.T
  wrt = W_r.T

  out = pl.pallas_call(
      _tc_body,
      grid=(N_BLOCKS,),
      in_specs=[
          pl.BlockSpec((NC, ROW_BLK, D), lambda i: (0, i, 0)),
          pl.BlockSpec((NW, ROW_BLK), lambda i: (0, i)),
          pl.BlockSpec((ROW_BLK, D), lambda i: (i, 0)),
          pl.BlockSpec((D, H), lambda i: (0, 0)),
          pl.BlockSpec((D, H), lambda i: (0, 0)),
          pl.BlockSpec((H,), lambda i: (0,)),
          pl.BlockSpec((H, 1280), lambda i: (0, 0)),
          pl.BlockSpec((1280,), lambda i: (0,)),
          pl.BlockSpec((1280, 480), lambda i: (0, 0)),
          pl.BlockSpec((480,), lambda i: (0,)),
          pl.BlockSpec((480,), lambda i: (0,)),
          pl.BlockSpec((1,), lambda i: (0,)),
      ],
      out_specs=pl.BlockSpec((ROW_BLK,), lambda i: (i,)),
      out_shape=jax.ShapeDtypeStruct((N_ACC,), jnp.float32),
  )(psum, pcnt, x_B1, wlt, wrt, b_l, w1f, b1f, w2f, b2f, w3r, b3)

  return out[:N_B1]


# speedup vs baseline: 1.0177x; 1.0015x over previous
import functools

import jax
import jax.numpy as jnp
from jax import lax
from jax.experimental import pallas as pl
from jax.experimental.pallas import tpu as pltpu
from jax.experimental.pallas import tpu_sc as plsc

N_B2 = 10000
N_B1 = 10000
E = 320000
D = 128
H = 128

NC = 2
NS = 16
LANES = 16
NW = NC * NS
CHUNK = 125
CHUNKS_PER_W = 80
IBLK = 8
NBLK = CHUNKS_PER_W // IBLK
N_ACC = 10240
ROWS_PER_TILE = N_ACC // NS
ZCOPY = 80
ROW_BLK = 512
N_BLOCKS = N_ACC // ROW_BLK


def _sc_segment_sum(x_b2, src_w, dst_w):
  mesh = plsc.VectorSubcoreMesh(core_axis_name="c", subcore_axis_name="s")

  @functools.partial(
      pl.kernel,
      out_type=(
          jax.ShapeDtypeStruct((NC, N_ACC, D), jnp.float32),
          jax.ShapeDtypeStruct((NW, N_ACC), jnp.float32),
      ),
      mesh=mesh,
      compiler_params=pltpu.CompilerParams(needs_layout_passes=False,
                                           use_tc_tiling_on_sc=True),
      scratch_types=[
          pltpu.VMEM((2, IBLK, CHUNK), jnp.int32),
          pltpu.VMEM((2, IBLK, CHUNK), jnp.int32),
          pltpu.VMEM((CHUNK, D), jnp.float32),
          pltpu.VMEM((CHUNK, D), jnp.float32),
          pltpu.VMEM((N_ACC,), jnp.float32),
          pltpu.VMEM_SHARED((N_ACC, D), jnp.float32),
          pltpu.SemaphoreType.DMA,
          pltpu.SemaphoreType.DMA,
          pltpu.SemaphoreType.DMA,
          pltpu.SemaphoreType.DMA,
          pltpu.SemaphoreType.DMA,
          pltpu.SemaphoreType.DMA,
      ],
  )
  def k(x_hbm, src_hbm, dst_hbm, psum_hbm, pcnt_hbm,
        src_b, dst_b, rows0_v, rows1_v, cnt_v, acc_sh,
        gsem0, gsem1, ssem0, ssem1, dsem0, dsem1):
    rows = (rows0_v, rows1_v)
    gsems = (gsem0, gsem1)
    ssems = (ssem0, ssem1)
    dsems = (dsem0, dsem1)
    c = lax.axis_index("c")
    s = lax.axis_index("s")
    wid = s * NC + c

    def start_iblk(bb, pb):
      pltpu.async_copy(src_hbm.at[wid, pl.ds(bb * IBLK, IBLK)], src_b.at[pb],
                       ssems[pb])
      pltpu.async_copy(dst_hbm.at[wid, pl.ds(bb * IBLK, IBLK)], dst_b.at[pb],
                       dsems[pb])

    def wait_iblk(pb):
      pltpu.make_async_copy(src_hbm.at[wid, pl.ds(0, IBLK)], src_b.at[pb],
                            ssems[pb]).wait()
      pltpu.make_async_copy(dst_hbm.at[wid, pl.ds(0, IBLK)], dst_b.at[pb],
                            dsems[pb]).wait()

    start_iblk(0, 0)

    zero16 = jnp.zeros((LANES,), jnp.float32)
    one16 = jnp.ones((LANES,), jnp.float32)
    tail_mask = lax.iota(jnp.int32, LANES) >= (LANES - (CHUNK % LANES))

    def zb(t, _):
      rows0_v[t // 8, pl.ds((t % 8) * LANES, LANES)] = zero16
      return 0
    lax.fori_loop(0, CHUNK * 8, zb, 0)

    def zc(t, _):
      cnt_v[pl.ds(t * LANES, LANES)] = zero16
      return 0
    lax.fori_loop(0, N_ACC // LANES, zc, 0)

    for kk in range(ROWS_PER_TILE // ZCOPY):
      pltpu.sync_copy(rows0_v.at[pl.ds(0, ZCOPY)],
                      acc_sh.at[pl.ds(s * ROWS_PER_TILE + kk * ZCOPY, ZCOPY)])
    plsc.subcore_barrier()

    wait_iblk(0)
    pltpu.async_copy(x_hbm.at[src_b.at[0, 0]], rows[0], gsems[0])

    def outer(bb2, _):
      for pb in range(2):
        bb = 2 * bb2 + pb
        np_ = 1 - pb

        @pl.when(bb + 1 < NBLK)
        def _():
          start_iblk(bb + 1, np_)

        for jj in range(IBLK):
          b = jj % 2
          nb = 1 - b
          if jj < IBLK - 1:
            pltpu.async_copy(x_hbm.at[src_b.at[pb, jj + 1]], rows[nb],
                             gsems[nb])
          else:
            @pl.when(bb + 1 < NBLK)
            def _():
              wait_iblk(np_)
              pltpu.async_copy(x_hbm.at[src_b.at[np_, 0]], rows[nb],
                               gsems[nb])
          for i in range(CHUNK // LANES):
            idx = dst_b[pb, jj, pl.ds(i * LANES, LANES)]
            plsc.addupdate_scatter(cnt_v, [idx], one16)
          idx = dst_b[pb, jj, pl.ds(CHUNK - LANES, LANES)]
          plsc.addupdate_scatter(cnt_v, [idx], one16, mask=tail_mask)

          pltpu.make_async_copy(x_hbm.at[src_b.at[pb, jj]], rows[b],
                                gsems[b]).wait()
          pltpu.sync_copy(rows[b], acc_sh.at[dst_b.at[pb, jj]], add=True)
      return 0
    lax.fori_loop(0, NBLK // 2, outer, 0)

    plsc.subcore_barrier()

    pltpu.sync_copy(acc_sh.at[pl.ds(s * ROWS_PER_TILE, ROWS_PER_TILE)],
                    psum_hbm.at[c, pl.ds(s * ROWS_PER_TILE, ROWS_PER_TILE)])
    pltpu.sync_copy(cnt_v, pcnt_hbm.at[wid])

  return k(x_b2, src_w, dst_w)


def _tc_body(psum_ref, pcnt_ref, x_ref, wl_ref, wr_ref, bl_ref,
             w1_ref, b1_ref, w2_ref, b2_ref, w3_ref, b3_ref, out_ref):
  summed = psum_ref[0] + psum_ref[1]
  cnt = jnp.sum(pcnt_ref[...], axis=0)
  mean = summed / jnp.maximum(cnt, 1.0)[:, None]
  h = (jnp.dot(mean, wl_ref[...], preferred_element_type=jnp.float32)
       + jnp.dot(x_ref[...], wr_ref[...], preferred_element_type=jnp.float32)
       + bl_ref[...])
  h = jnp.dot(h, w1_ref[...], preferred_element_type=jnp.float32) + b1_ref[...]
  h = jnp.where(h > 0, h, 0.01 * h)
  h = jnp.dot(h, w2_ref[...], preferred_element_type=jnp.float32) + b2_ref[...]
  h = jnp.where(h > 0, h, 0.01 * h)
  out_ref[...] = jnp.sum(h * w3_ref[...], axis=1) + b3_ref[...]


def kernel(x_B2, x_B1, edge_index, W_l, b_l, W_r, W1, b1, g1, be1, W2, b2, g2, be2, W3, b3):
  if edge_index.dtype == jnp.int64:
    ei32 = lax.bitcast_convert_type(edge_index, jnp.int32)[:, :, 0]
  else:
    ei32 = edge_index.astype(jnp.int32)
  src_w = ei32[0].reshape(NW, CHUNKS_PER_W, CHUNK)
  dst_w = ei32[1].reshape(NW, CHUNKS_PER_W, CHUNK)

  psum, pcnt = _sc_segment_sum(x_B2, src_w, dst_w)

  eps = 1e-5
  s1 = g1 / jnp.sqrt(1.0 + eps)
  s2 = g2 / jnp.sqrt(1.0 + eps)
  w1f = (W1 * s1[:, None]).T
  b1f = b1 * s1 + be1
  w2f = (W2 * s2[:, None]).T
  b2f = b2 * s2 + be2
  w3r = W3[0]
  wlt = W_l.T
  wrt = W_r.T

  out = pl.pallas_call(
      _tc_body,
      grid=(N_BLOCKS,),
      in_specs=[
          pl.BlockSpec((NC, ROW_BLK, D), lambda i: (0, i, 0)),
          pl.BlockSpec((NW, ROW_BLK), lambda i: (0, i)),
          pl.BlockSpec((ROW_BLK, D), lambda i: (i, 0)),
          pl.BlockSpec((D, H), lambda i: (0, 0)),
          pl.BlockSpec((D, H), lambda i: (0, 0)),
          pl.BlockSpec((H,), lambda i: (0,)),
          pl.BlockSpec((H, 1280), lambda i: (0, 0)),
          pl.BlockSpec((1280,), lambda i: (0,)),
          pl.BlockSpec((1280, 480), lambda i: (0, 0)),
          pl.BlockSpec((480,), lambda i: (0,)),
          pl.BlockSpec((480,), lambda i: (0,)),
          pl.BlockSpec((1,), lambda i: (0,)),
      ],
      out_specs=pl.BlockSpec((ROW_BLK,), lambda i: (i,)),
      out_shape=jax.ShapeDtypeStruct((N_ACC,), jnp.float32),
  )(psum, pcnt, x_B1, wlt, wrt, b_l, w1f, b1f, w2f, b2f, w3r, b3)

  return out[:N_B1]


# speedup vs baseline: 1.2029x; 1.1819x over previous
import functools

import jax
import jax.numpy as jnp
from jax import lax
from jax.experimental import pallas as pl
from jax.experimental.pallas import tpu as pltpu
from jax.experimental.pallas import tpu_sc as plsc

N_B2 = 10000
N_B1 = 10000
E = 320000
D = 128
H = 128

NC = 2
NS = 16
LANES = 16
NW = NC * NS
CHUNK = 128
E_PER_W = 10240
E_LAST = E - (NW - 1) * E_PER_W
IBLK = 4
NBLK = E_PER_W // (IBLK * CHUNK)
NBLK_LAST = E_LAST // (IBLK * CHUNK)
N_ACC = 10240
ROWS_PER_TILE = N_ACC // NS
ZCOPY = 80
ROW_BLK = 1024
N_BLOCKS = N_ACC // ROW_BLK


def _sc_segment_sum(x_b2, ei):
  mesh = plsc.VectorSubcoreMesh(core_axis_name="c", subcore_axis_name="s")

  @functools.partial(
      pl.kernel,
      out_type=(
          jax.ShapeDtypeStruct((NC, N_ACC, D), jnp.float32),
          jax.ShapeDtypeStruct((NW, N_ACC), jnp.float32),
      ),
      mesh=mesh,
      compiler_params=pltpu.CompilerParams(needs_layout_passes=False,
                                           use_tc_tiling_on_sc=True),
      scratch_types=[
          pltpu.VMEM((2, 2, IBLK * CHUNK), jnp.int32),
          pltpu.VMEM((CHUNK, D), jnp.float32),
          pltpu.VMEM((CHUNK, D), jnp.float32),
          pltpu.VMEM((N_ACC,), jnp.float32),
          pltpu.VMEM_SHARED((N_ACC, D), jnp.float32),
          pltpu.SemaphoreType.DMA,
          pltpu.SemaphoreType.DMA,
          pltpu.SemaphoreType.DMA,
          pltpu.SemaphoreType.DMA,
          pltpu.SemaphoreType.DMA,
          pltpu.SemaphoreType.DMA,
      ],
  )
  def k(x_hbm, ei_hbm, psum_hbm, pcnt_hbm,
        eib, rows0_v, rows1_v, cnt_v, acc_sh,
        gsem0, gsem1, isem0, isem1, ksem0, ksem1):
    rows = (rows0_v, rows1_v)
    gsems = (gsem0, gsem1)
    isems = (isem0, isem1)
    ksems = (ksem0, ksem1)
    c = lax.axis_index("c")
    s = lax.axis_index("s")
    wid = s * NC + c
    base = wid * E_PER_W
    my_nblk = jnp.where(wid == NW - 1, NBLK_LAST, NBLK)

    def start_iblk(bb, pb):
      pltpu.async_copy(
          ei_hbm.at[:, pl.ds(base + bb * IBLK * CHUNK, IBLK * CHUNK)],
          eib.at[pb], isems[pb])

    def wait_iblk(pb):
      pltpu.make_async_copy(ei_hbm.at[:, pl.ds(0, IBLK * CHUNK)],
                            eib.at[pb], isems[pb]).wait()

    start_iblk(0, 0)

    zero16 = jnp.zeros((LANES,), jnp.float32)
    one16 = jnp.ones((LANES,), jnp.float32)

    def zb(t, _):
      for i in range(8):
        rows0_v[t, pl.ds(i * LANES, LANES)] = zero16
      return 0
    lax.fori_loop(0, ZCOPY, zb, 0)

    def zc(t, _):
      for i in range(16):
        cnt_v[pl.ds((t * 16 + i) * LANES, LANES)] = zero16
      return 0
    lax.fori_loop(0, N_ACC // LANES // 16, zc, 0)

    for kk in range(ROWS_PER_TILE // ZCOPY):
      pltpu.async_copy(rows0_v.at[pl.ds(0, ZCOPY)],
                       acc_sh.at[pl.ds(s * ROWS_PER_TILE + kk * ZCOPY, ZCOPY)],
                       gsem0)
    for kk in range(ROWS_PER_TILE // ZCOPY):
      pltpu.make_async_copy(
          rows0_v.at[pl.ds(0, ZCOPY)],
          acc_sh.at[pl.ds(s * ROWS_PER_TILE + kk * ZCOPY, ZCOPY)],
          gsem0).wait()
    plsc.subcore_barrier()

    wait_iblk(0)
    pltpu.async_copy(x_hbm.at[eib.at[0, 0, pl.ds(0, CHUNK)]], rows[0],
                     gsems[0])

    def outer(bb2, _):
      for pb in range(2):
        bb = 2 * bb2 + pb
        np_ = 1 - pb

        @pl.when(bb + 1 < my_nblk)
        def _():
          start_iblk(bb + 1, np_)

        for jj in range(IBLK):
          b = jj % 2
          nb = 1 - b
          j = bb * IBLK + jj

          @pl.when(bb < my_nblk)
          def _():
            if jj < IBLK - 1:
              @pl.when(j >= 1)
              def _():
                pltpu.make_async_copy(
                    rows[nb], acc_sh.at[eib.at[pb, 1, pl.ds(0, CHUNK)]],
                    ksems[nb]).wait()
              pltpu.async_copy(
                  x_hbm.at[eib.at[pb, 0, pl.ds((jj + 1) * CHUNK, CHUNK)]],
                  rows[nb], gsems[nb])
            else:
              @pl.when(bb + 1 < my_nblk)
              def _():
                pltpu.make_async_copy(
                    rows[nb], acc_sh.at[eib.at[pb, 1, pl.ds(0, CHUNK)]],
                    ksems[nb]).wait()
                wait_iblk(np_)
                pltpu.async_copy(
                    x_hbm.at[eib.at[np_, 0, pl.ds(0, CHUNK)]],
                    rows[nb], gsems[nb])
            for i in range(CHUNK // LANES):
              idx = eib[pb, 1, pl.ds(jj * CHUNK + i * LANES, LANES)]
              plsc.addupdate_scatter(cnt_v, [idx], one16)

            pltpu.make_async_copy(
                x_hbm.at[eib.at[pb, 0, pl.ds(jj * CHUNK, CHUNK)]],
                rows[b], gsems[b]).wait()
            pltpu.async_copy(rows[b],
                             acc_sh.at[eib.at[pb, 1,
                                              pl.ds(jj * CHUNK, CHUNK)]],
                             ksems[b], add=True)
      return 0
    lax.fori_loop(0, NBLK // 2, outer, 0)

    pltpu.make_async_copy(rows[0], acc_sh.at[eib.at[0, 1, pl.ds(0, CHUNK)]],
                          ksems[0]).wait()
    pltpu.make_async_copy(rows[1], acc_sh.at[eib.at[0, 1, pl.ds(0, CHUNK)]],
                          ksems[1]).wait()
    plsc.subcore_barrier()

    pltpu.async_copy(acc_sh.at[pl.ds(s * ROWS_PER_TILE, ROWS_PER_TILE)],
                     psum_hbm.at[c, pl.ds(s * ROWS_PER_TILE, ROWS_PER_TILE)],
                     gsem0)
    pltpu.async_copy(cnt_v, pcnt_hbm.at[wid], gsem1)
    pltpu.make_async_copy(acc_sh.at[pl.ds(s * ROWS_PER_TILE, ROWS_PER_TILE)],
                          psum_hbm.at[c, pl.ds(s * ROWS_PER_TILE,
                                               ROWS_PER_TILE)], gsem0).wait()
    pltpu.make_async_copy(cnt_v, pcnt_hbm.at[wid], gsem1).wait()

  return k(x_b2, ei)


def _tc_body(psum_ref, pcnt_ref, x_ref, wl_ref, wr_ref, bl_ref,
             w1_ref, b1_ref, w2_ref, b2_ref, w3_ref, b3_ref, out_ref):
  summed = psum_ref[0] + psum_ref[1]
  cnt = jnp.sum(pcnt_ref[...], axis=0)
  mean = summed / jnp.maximum(cnt, 1.0)[:, None]
  h = (jnp.dot(mean, wl_ref[...], preferred_element_type=jnp.float32)
       + jnp.dot(x_ref[...], wr_ref[...], preferred_element_type=jnp.float32)
       + bl_ref[...])
  h = jnp.dot(h, w1_ref[...], preferred_element_type=jnp.float32) + b1_ref[...]
  h = jnp.where(h > 0, h, 0.01 * h)
  h = jnp.dot(h, w2_ref[...], preferred_element_type=jnp.float32) + b2_ref[...]
  h = jnp.where(h > 0, h, 0.01 * h)
  r = jnp.dot(h, w3_ref[...], preferred_element_type=jnp.float32)
  out_ref[...] = r[:, 0] + b3_ref[...]


def kernel(x_B2, x_B1, edge_index, W_l, b_l, W_r, W1, b1, g1, be1, W2, b2, g2, be2, W3, b3):
  ei = edge_index.astype(jnp.int32)

  psum, pcnt = _sc_segment_sum(x_B2, ei)

  eps = 1e-5
  s1 = g1 / jnp.sqrt(1.0 + eps)
  s2 = g2 / jnp.sqrt(1.0 + eps)
  w1f = (W1 * s1[:, None]).T
  b1f = b1 * s1 + be1
  w2f = (W2 * s2[:, None]).T
  b2f = b2 * s2 + be2
  w3t = W3.T
  wlt = W_l.T
  wrt = W_r.T

  out = pl.pallas_call(
      _tc_body,
      grid=(N_BLOCKS,),
      in_specs=[
          pl.BlockSpec((NC, ROW_BLK, D), lambda i: (0, i, 0)),
          pl.BlockSpec((NW, ROW_BLK), lambda i: (0, i)),
          pl.BlockSpec((ROW_BLK, D), lambda i: (i, 0)),
          pl.BlockSpec((D, H), lambda i: (0, 0)),
          pl.BlockSpec((D, H), lambda i: (0, 0)),
          pl.BlockSpec((H,), lambda i: (0,)),
          pl.BlockSpec((H, 1280), lambda i: (0, 0)),
          pl.BlockSpec((1280,), lambda i: (0,)),
          pl.BlockSpec((1280, 480), lambda i: (0, 0)),
          pl.BlockSpec((480,), lambda i: (0,)),
          pl.BlockSpec((480, 1), lambda i: (0, 0)),
          pl.BlockSpec((1,), lambda i: (0,)),
      ],
      out_specs=pl.BlockSpec((ROW_BLK,), lambda i: (i,)),
      out_shape=jax.ShapeDtypeStruct((N_ACC,), jnp.float32),
  )(psum, pcnt, x_B1, wlt, wrt, b_l, w1f, b1f, w2f, b2f, w3t, b3)

  return out[:N_B1]


# speedup vs baseline: 1.2056x; 1.0022x over previous
import functools

import jax
import jax.numpy as jnp
from jax import lax
from jax.experimental import pallas as pl
from jax.experimental.pallas import tpu as pltpu
from jax.experimental.pallas import tpu_sc as plsc

N_B2 = 10000
N_B1 = 10000
E = 320000
D = 128
H = 128

NC = 2
NS = 16
LANES = 16
NW = NC * NS
CHUNK = 128
E_PER_W = 10240
E_LAST = E - (NW - 1) * E_PER_W
IBLK = 4
NBLK = E_PER_W // (IBLK * CHUNK)
NBLK_LAST = E_LAST // (IBLK * CHUNK)
N_ACC = 10240
ROWS_PER_TILE = N_ACC // NS
ZCOPY = 80
ROW_BLK = 2048
N_BLOCKS = -(-N_B1 // ROW_BLK)


def _sc_segment_sum(x_b2, ei):
  mesh = plsc.VectorSubcoreMesh(core_axis_name="c", subcore_axis_name="s")

  @functools.partial(
      pl.kernel,
      out_type=(
          jax.ShapeDtypeStruct((NC, N_ACC, D), jnp.float32),
          jax.ShapeDtypeStruct((NW, N_ACC), jnp.float32),
      ),
      mesh=mesh,
      compiler_params=pltpu.CompilerParams(needs_layout_passes=False,
                                           use_tc_tiling_on_sc=True),
      scratch_types=[
          pltpu.VMEM((2, 2, IBLK * CHUNK), jnp.int32),
          pltpu.VMEM((CHUNK, D), jnp.float32),
          pltpu.VMEM((CHUNK, D), jnp.float32),
          pltpu.VMEM((N_ACC,), jnp.float32),
          pltpu.VMEM_SHARED((N_ACC, D), jnp.float32),
          pltpu.SemaphoreType.DMA,
          pltpu.SemaphoreType.DMA,
          pltpu.SemaphoreType.DMA,
          pltpu.SemaphoreType.DMA,
          pltpu.SemaphoreType.DMA,
          pltpu.SemaphoreType.DMA,
      ],
  )
  def k(x_hbm, ei_hbm, psum_hbm, pcnt_hbm,
        eib, rows0_v, rows1_v, cnt_v, acc_sh,
        gsem0, gsem1, isem0, isem1, ksem0, ksem1):
    rows = (rows0_v, rows1_v)
    gsems = (gsem0, gsem1)
    isems = (isem0, isem1)
    ksems = (ksem0, ksem1)
    c = lax.axis_index("c")
    s = lax.axis_index("s")
    wid = s * NC + c
    base = wid * E_PER_W
    my_nblk = jnp.where(wid == NW - 1, NBLK_LAST, NBLK)

    def start_iblk(bb, pb):
      pltpu.async_copy(
          ei_hbm.at[:, pl.ds(base + bb * IBLK * CHUNK, IBLK * CHUNK)],
          eib.at[pb], isems[pb])

    def wait_iblk(pb):
      pltpu.make_async_copy(ei_hbm.at[:, pl.ds(0, IBLK * CHUNK)],
                            eib.at[pb], isems[pb]).wait()

    start_iblk(0, 0)

    zero16 = jnp.zeros((LANES,), jnp.float32)
    one16 = jnp.ones((LANES,), jnp.float32)

    def zb(t, _):
      for i in range(8):
        rows0_v[t, pl.ds(i * LANES, LANES)] = zero16
      return 0
    lax.fori_loop(0, ZCOPY, zb, 0)

    def zc(t, _):
      for i in range(16):
        cnt_v[pl.ds((t * 16 + i) * LANES, LANES)] = zero16
      return 0
    lax.fori_loop(0, N_ACC // LANES // 16, zc, 0)

    for kk in range(ROWS_PER_TILE // ZCOPY):
      pltpu.async_copy(rows0_v.at[pl.ds(0, ZCOPY)],
                       acc_sh.at[pl.ds(s * ROWS_PER_TILE + kk * ZCOPY, ZCOPY)],
                       gsem0)
    for kk in range(ROWS_PER_TILE // ZCOPY):
      pltpu.make_async_copy(
          rows0_v.at[pl.ds(0, ZCOPY)],
          acc_sh.at[pl.ds(s * ROWS_PER_TILE + kk * ZCOPY, ZCOPY)],
          gsem0).wait()
    plsc.subcore_barrier()

    wait_iblk(0)
    pltpu.async_copy(x_hbm.at[eib.at[0, 0, pl.ds(0, CHUNK)]], rows[0],
                     gsems[0])

    def outer(bb2, _):
      for pb in range(2):
        bb = 2 * bb2 + pb
        np_ = 1 - pb

        @pl.when(bb + 1 < my_nblk)
        def _():
          start_iblk(bb + 1, np_)

        for jj in range(IBLK):
          b = jj % 2
          nb = 1 - b
          j = bb * IBLK + jj

          @pl.when(bb < my_nblk)
          def _():
            if jj < IBLK - 1:
              @pl.when(j >= 1)
              def _():
                pltpu.make_async_copy(
                    rows[nb], acc_sh.at[eib.at[pb, 1, pl.ds(0, CHUNK)]],
                    ksems[nb]).wait()
              pltpu.async_copy(
                  x_hbm.at[eib.at[pb, 0, pl.ds((jj + 1) * CHUNK, CHUNK)]],
                  rows[nb], gsems[nb])
            else:
              @pl.when(bb + 1 < my_nblk)
              def _():
                pltpu.make_async_copy(
                    rows[nb], acc_sh.at[eib.at[pb, 1, pl.ds(0, CHUNK)]],
                    ksems[nb]).wait()
                wait_iblk(np_)
                pltpu.async_copy(
                    x_hbm.at[eib.at[np_, 0, pl.ds(0, CHUNK)]],
                    rows[nb], gsems[nb])
            for i in range(CHUNK // LANES):
              idx = eib[pb, 1, pl.ds(jj * CHUNK + i * LANES, LANES)]
              plsc.addupdate_scatter(cnt_v, [idx], one16)

            pltpu.make_async_copy(
                x_hbm.at[eib.at[pb, 0, pl.ds(jj * CHUNK, CHUNK)]],
                rows[b], gsems[b]).wait()
            pltpu.async_copy(rows[b],
                             acc_sh.at[eib.at[pb, 1,
                                              pl.ds(jj * CHUNK, CHUNK)]],
                             ksems[b], add=True)
      return 0
    lax.fori_loop(0, NBLK // 2, outer, 0)

    pltpu.make_async_copy(rows[0], acc_sh.at[eib.at[0, 1, pl.ds(0, CHUNK)]],
                          ksems[0]).wait()
    pltpu.make_async_copy(rows[1], acc_sh.at[eib.at[0, 1, pl.ds(0, CHUNK)]],
                          ksems[1]).wait()
    plsc.subcore_barrier()

    pltpu.async_copy(acc_sh.at[pl.ds(s * ROWS_PER_TILE, ROWS_PER_TILE)],
                     psum_hbm.at[c, pl.ds(s * ROWS_PER_TILE, ROWS_PER_TILE)],
                     gsem0)
    pltpu.async_copy(cnt_v, pcnt_hbm.at[wid], gsem1)
    pltpu.make_async_copy(acc_sh.at[pl.ds(s * ROWS_PER_TILE, ROWS_PER_TILE)],
                          psum_hbm.at[c, pl.ds(s * ROWS_PER_TILE,
                                               ROWS_PER_TILE)], gsem0).wait()
    pltpu.make_async_copy(cnt_v, pcnt_hbm.at[wid], gsem1).wait()

  return k(x_b2, ei)


def _tc_body(psum_ref, pcnt_ref, x_ref, wl_ref, wr_ref, bl_ref,
             w1_ref, b1_ref, w2_ref, b2_ref, w3_ref, b3_ref, out_ref):
  summed = psum_ref[0] + psum_ref[1]
  cnt = jnp.sum(pcnt_ref[...], axis=0)
  mean = summed / jnp.maximum(cnt, 1.0)[:, None]
  h = (jnp.dot(mean, wl_ref[...], preferred_element_type=jnp.float32)
       + jnp.dot(x_ref[...], wr_ref[...], preferred_element_type=jnp.float32)
       + bl_ref[...])
  h = jnp.dot(h, w1_ref[...], preferred_element_type=jnp.float32) + b1_ref[...]
  h = jnp.where(h > 0, h, 0.01 * h)
  h = jnp.dot(h, w2_ref[...], preferred_element_type=jnp.float32) + b2_ref[...]
  h = jnp.where(h > 0, h, 0.01 * h)
  r = jnp.dot(h, w3_ref[...], preferred_element_type=jnp.float32)
  out_ref[...] = r[:, 0] + b3_ref[...]


def kernel(x_B2, x_B1, edge_index, W_l, b_l, W_r, W1, b1, g1, be1, W2, b2, g2, be2, W3, b3):
  ei = edge_index.astype(jnp.int32)

  psum, pcnt = _sc_segment_sum(x_B2, ei)

  eps = 1e-5
  s1 = g1 / jnp.sqrt(1.0 + eps)
  s2 = g2 / jnp.sqrt(1.0 + eps)
  w1f = (W1 * s1[:, None]).T
  b1f = b1 * s1 + be1
  w2f = (W2 * s2[:, None]).T
  b2f = b2 * s2 + be2
  w3t = W3.T
  wlt = W_l.T
  wrt = W_r.T

  out = pl.pallas_call(
      _tc_body,
      grid=(N_BLOCKS,),
      in_specs=[
          pl.BlockSpec((NC, ROW_BLK, D), lambda i: (0, i, 0)),
          pl.BlockSpec((NW, ROW_BLK), lambda i: (0, i)),
          pl.BlockSpec((ROW_BLK, D), lambda i: (i, 0)),
          pl.BlockSpec((D, H), lambda i: (0, 0)),
          pl.BlockSpec((D, H), lambda i: (0, 0)),
          pl.BlockSpec((H,), lambda i: (0,)),
          pl.BlockSpec((H, 1280), lambda i: (0, 0)),
          pl.BlockSpec((1280,), lambda i: (0,)),
          pl.BlockSpec((1280, 480), lambda i: (0, 0)),
          pl.BlockSpec((480,), lambda i: (0,)),
          pl.BlockSpec((480, 1), lambda i: (0, 0)),
          pl.BlockSpec((1,), lambda i: (0,)),
      ],
      out_specs=pl.BlockSpec((ROW_BLK,), lambda i: (i,)),
      out_shape=jax.ShapeDtypeStruct((N_B1,), jnp.float32),
  )(psum, pcnt, x_B1, wlt, wrt, b_l, w1f, b1f, w2f, b2f, w3t, b3)

  return out


# speedup vs baseline: 1.3080x; 1.0849x over previous
import functools

import jax
import jax.numpy as jnp
from jax import lax
from jax.experimental import pallas as pl
from jax.experimental.pallas import tpu as pltpu
from jax.experimental.pallas import tpu_sc as plsc

N_B2 = 10000
N_B1 = 10000
E = 320000
D = 128
H = 128

NC = 2
NS = 16
LANES = 16
NW = NC * NS
CHUNK = 64
NBUF = 4
E_PER_W = 10240
E_LAST = E - (NW - 1) * E_PER_W
IBLK = 8
NBLK = E_PER_W // (IBLK * CHUNK)
NBLK_LAST = E_LAST // (IBLK * CHUNK)
N_ACC = 10240
ROWS_PER_TILE = N_ACC // NS
ZCOPY = 80
ROW_BLK = 2048
N_BLOCKS = -(-N_B1 // ROW_BLK)


def _sc_segment_sum(x_b2, ei):
  mesh = plsc.VectorSubcoreMesh(core_axis_name="c", subcore_axis_name="s")

  @functools.partial(
      pl.kernel,
      out_type=(
          jax.ShapeDtypeStruct((NC, N_ACC, D), jnp.float32),
          jax.ShapeDtypeStruct((NW, N_ACC), jnp.float32),
      ),
      mesh=mesh,
      compiler_params=pltpu.CompilerParams(needs_layout_passes=False,
                                           use_tc_tiling_on_sc=True),
      scratch_types=[
          pltpu.VMEM((2, 2, IBLK * CHUNK), jnp.int32),
          pltpu.VMEM((CHUNK, D), jnp.float32),
          pltpu.VMEM((CHUNK, D), jnp.float32),
          pltpu.VMEM((CHUNK, D), jnp.float32),
          pltpu.VMEM((CHUNK, D), jnp.float32),
          pltpu.VMEM((N_ACC,), jnp.float32),
          pltpu.VMEM_SHARED((N_ACC, D), jnp.float32),
          pltpu.SemaphoreType.DMA,
          pltpu.SemaphoreType.DMA,
          pltpu.SemaphoreType.DMA,
          pltpu.SemaphoreType.DMA,
          pltpu.SemaphoreType.DMA,
          pltpu.SemaphoreType.DMA,
          pltpu.SemaphoreType.DMA,
          pltpu.SemaphoreType.DMA,
          pltpu.SemaphoreType.DMA,
          pltpu.SemaphoreType.DMA,
      ],
  )
  def k(x_hbm, ei_hbm, psum_hbm, pcnt_hbm,
        eib, rows0_v, rows1_v, rows2_v, rows3_v, cnt_v, acc_sh,
        gsem0, gsem1, gsem2, gsem3, isem0, isem1,
        ksem0, ksem1, ksem2, ksem3):
    rows = (rows0_v, rows1_v, rows2_v, rows3_v)
    gsems = (gsem0, gsem1, gsem2, gsem3)
    isems = (isem0, isem1)
    ksems = (ksem0, ksem1, ksem2, ksem3)
    c = lax.axis_index("c")
    s = lax.axis_index("s")
    wid = s * NC + c
    base = wid * E_PER_W
    my_nblk = jnp.where(wid == NW - 1, NBLK_LAST, NBLK)

    def start_iblk(bb, pb):
      pltpu.async_copy(
          ei_hbm.at[:, pl.ds(base + bb * IBLK * CHUNK, IBLK * CHUNK)],
          eib.at[pb], isems[pb])

    def wait_iblk(pb):
      pltpu.make_async_copy(ei_hbm.at[:, pl.ds(0, IBLK * CHUNK)],
                            eib.at[pb], isems[pb]).wait()

    start_iblk(0, 0)

    zero16 = jnp.zeros((LANES,), jnp.float32)
    one16 = jnp.ones((LANES,), jnp.float32)

    def zb(t, _):
      for i in range(8):
        rows0_v[t, pl.ds(i * LANES, LANES)] = zero16
      return 0
    lax.fori_loop(0, ZCOPY, zb, 0)

    def zc(t, _):
      for i in range(16):
        cnt_v[pl.ds((t * 16 + i) * LANES, LANES)] = zero16
      return 0
    lax.fori_loop(0, N_ACC // LANES // 16, zc, 0)

    for kk in range(ROWS_PER_TILE // ZCOPY):
      pltpu.async_copy(rows0_v.at[pl.ds(0, ZCOPY)],
                       acc_sh.at[pl.ds(s * ROWS_PER_TILE + kk * ZCOPY, ZCOPY)],
                       gsem0)
    for kk in range(ROWS_PER_TILE // ZCOPY):
      pltpu.make_async_copy(
          rows0_v.at[pl.ds(0, ZCOPY)],
          acc_sh.at[pl.ds(s * ROWS_PER_TILE + kk * ZCOPY, ZCOPY)],
          gsem0).wait()
    plsc.subcore_barrier()

    wait_iblk(0)
    for pj in range(3):
      pltpu.async_copy(x_hbm.at[eib.at[0, 0, pl.ds(pj * CHUNK, CHUNK)]],
                       rows[pj], gsems[pj])

    def outer(bb2, _):
      for pb in range(2):
        bb = 2 * bb2 + pb
        np_ = 1 - pb

        @pl.when(bb + 1 < my_nblk)
        def _():
          start_iblk(bb + 1, np_)

        for jj in range(IBLK):
          j = bb * IBLK + jj

          @pl.when(bb < my_nblk)
          def _():
            b = jj % NBUF
            nb = (jj + 3) % NBUF
            @pl.when(j >= 1)
            def _():
              pltpu.make_async_copy(
                  rows[nb], acc_sh.at[eib.at[pb, 1, pl.ds(0, CHUNK)]],
                  ksems[nb]).wait()
            if jj == IBLK - 3:
              @pl.when(bb + 1 < my_nblk)
              def _():
                wait_iblk(np_)
            if jj < IBLK - 3:
              @pl.when(j + 3 < my_nblk * IBLK)
              def _():
                pltpu.async_copy(
                    x_hbm.at[eib.at[pb, 0, pl.ds((jj + 3) * CHUNK, CHUNK)]],
                    rows[nb], gsems[nb])
            else:
              @pl.when(bb + 1 < my_nblk)
              def _():
                pltpu.async_copy(
                    x_hbm.at[eib.at[np_, 0,
                                    pl.ds(((jj + 3) % IBLK) * CHUNK, CHUNK)]],
                    rows[nb], gsems[nb])
            for i in range(CHUNK // LANES):
              idx = eib[pb, 1, pl.ds(jj * CHUNK + i * LANES, LANES)]
              plsc.addupdate_scatter(cnt_v, [idx], one16)

            pltpu.make_async_copy(
                x_hbm.at[eib.at[pb, 0, pl.ds(jj * CHUNK, CHUNK)]],
                rows[b], gsems[b]).wait()
            pltpu.async_copy(rows[b],
                             acc_sh.at[eib.at[pb, 1,
                                              pl.ds(jj * CHUNK, CHUNK)]],
                             ksems[b], add=True)
      return 0
    lax.fori_loop(0, NBLK // 2, outer, 0)

    pltpu.make_async_copy(rows[3], acc_sh.at[eib.at[0, 1, pl.ds(0, CHUNK)]],
                          ksems[3]).wait()
    plsc.subcore_barrier()

    pltpu.async_copy(acc_sh.at[pl.ds(s * ROWS_PER_TILE, ROWS_PER_TILE)],
                     psum_hbm.at[c, pl.ds(s * ROWS_PER_TILE, ROWS_PER_TILE)],
                     gsem0)
    pltpu.async_copy(cnt_v, pcnt_hbm.at[wid], gsem1)
    pltpu.make_async_copy(acc_sh.at[pl.ds(s * ROWS_PER_TILE, ROWS_PER_TILE)],
                          psum_hbm.at[c, pl.ds(s * ROWS_PER_TILE,
                                               ROWS_PER_TILE)], gsem0).wait()
    pltpu.make_async_copy(cnt_v, pcnt_hbm.at[wid], gsem1).wait()

  return k(x_b2, ei)


def _tc_body(psum_ref, pcnt_ref, x_ref, wl_ref, wr_ref, bl_ref,
             w1_ref, b1_ref, w2_ref, b2_ref, w3_ref, b3_ref, out_ref):
  summed = psum_ref[0] + psum_ref[1]
  cnt = jnp.sum(pcnt_ref[...], axis=0)
  mean = summed / jnp.maximum(cnt, 1.0)[:, None]
  h = (jnp.dot(mean, wl_ref[...], preferred_element_type=jnp.float32)
       + jnp.dot(x_ref[...], wr_ref[...], preferred_element_type=jnp.float32)
       + bl_ref[...])
  h = jnp.dot(h, w1_ref[...], preferred_element_type=jnp.float32) + b1_ref[...]
  h = jnp.where(h > 0, h, 0.01 * h)
  h = jnp.dot(h, w2_ref[...], preferred_element_type=jnp.float32) + b2_ref[...]
  h = jnp.where(h > 0, h, 0.01 * h)
  r = jnp.dot(h, w3_ref[...], preferred_element_type=jnp.float32)
  out_ref[...] = r[:, 0] + b3_ref[...]


def kernel(x_B2, x_B1, edge_index, W_l, b_l, W_r, W1, b1, g1, be1, W2, b2, g2, be2, W3, b3):
  ei = edge_index.astype(jnp.int32)

  psum, pcnt = _sc_segment_sum(x_B2, ei)

  eps = 1e-5
  s1 = g1 / jnp.sqrt(1.0 + eps)
  s2 = g2 / jnp.sqrt(1.0 + eps)
  w1f = (W1 * s1[:, None]).T
  b1f = b1 * s1 + be1
  w2f = (W2 * s2[:, None]).T
  b2f = b2 * s2 + be2
  w3t = W3.T
  wlt = W_l.T
  wrt = W_r.T

  out = pl.pallas_call(
      _tc_body,
      grid=(N_BLOCKS,),
      in_specs=[
          pl.BlockSpec((NC, ROW_BLK, D), lambda i: (0, i, 0)),
          pl.BlockSpec((NW, ROW_BLK), lambda i: (0, i)),
          pl.BlockSpec((ROW_BLK, D), lambda i: (i, 0)),
          pl.BlockSpec((D, H), lambda i: (0, 0)),
          pl.BlockSpec((D, H), lambda i: (0, 0)),
          pl.BlockSpec((H,), lambda i: (0,)),
          pl.BlockSpec((H, 1280), lambda i: (0, 0)),
          pl.BlockSpec((1280,), lambda i: (0,)),
          pl.BlockSpec((1280, 480), lambda i: (0, 0)),
          pl.BlockSpec((480,), lambda i: (0,)),
          pl.BlockSpec((480, 1), lambda i: (0, 0)),
          pl.BlockSpec((1,), lambda i: (0,)),
      ],
      out_specs=pl.BlockSpec((ROW_BLK,), lambda i: (i,)),
      out_shape=jax.ShapeDtypeStruct((N_B1,), jnp.float32),
  )(psum, pcnt, x_B1, wlt, wrt, b_l, w1f, b1f, w2f, b2f, w3t, b3)

  return out
